# baseline (speedup 1.0000x reference)
import jax
import jax.numpy as jnp
from jax.experimental import pallas as pl

_R = 128
_NSTEPS = 512 // _R
_NPIX = float(512 * 512)

_G = [0,
      (1 << 8) + (1 << 24),
      1 + (2 << 24),
      1 + (1 << 8) + (1 << 16) + (3 << 24)]
_H = [1,
      1 + (1 << 16),
      1 + (1 << 8),
      1 + (1 << 8) + (1 << 16) + (1 << 24)]


def _dice_body(o_ref, t_ref, acc_ref, score_ref):
    step = pl.program_id(0)

    o0 = o_ref[:, 0]
    o1 = o_ref[:, 1]
    o2 = o_ref[:, 2]
    o3 = o_ref[:, 3]
    t = t_ref[:, 0]

    gt1 = o1 > o0
    gt3 = o3 > o2
    gtb = jnp.maximum(o2, o3) > jnp.maximum(o0, o1)

    i32 = jnp.int32
    g = jnp.where(gtb,
                  jnp.where(gt3, i32(_G[3]), i32(_G[2])),
                  jnp.where(gt1, i32(_G[1]), i32(_G[0])))
    h = jnp.where(gtb,
                  jnp.where(gt3, i32(_H[3]), i32(_H[2])),
                  jnp.where(gt1, i32(_H[1]), i32(_H[0])))
    idx = g >> 24
    mp = jnp.where(idx == t, h, i32(0))
    ta = t >> 1
    tb = t & 1
    tab = ta & tb

    rg = jnp.sum(g, axis=1)
    rm = jnp.sum(mp, axis=1)
    rta = jnp.sum(ta, axis=1)
    rtb = jnp.sum(tb, axis=1)
    rtab = jnp.sum(tab, axis=1)

    m8 = i32(0xFF)
    part = [rg & m8, (rg >> 8) & m8, (rg >> 16) & m8,
            rta, rtb, rtab,
            rm & m8, (rm >> 8) & m8, (rm >> 16) & m8,
            (rm >> 24) & m8]

    @pl.when(step == 0)
    def _():
        acc_ref[...] = jnp.zeros_like(acc_ref)

    for k in range(10):
        acc_ref[k] += part[k]

    @pl.when(step == _NSTEPS - 1)
    def _():
        st = jnp.sum(acc_ref[...], axis=2).astype(jnp.float32)
        sa, sb, sab = st[0], st[1], st[2]
        sta, stb, stab = st[3], st[4], st[5]
        sm, sma, smb, smab = st[6], st[7], st[8], st[9]
        p3, p2, p1 = sab, sa - sab, sb - sab
        p0 = _NPIX - sa - sb + sab
        t3, t2, t1 = stab, sta - stab, stb - stab
        t0 = _NPIX - sta - stb + stab
        i3, i2, i1 = smab, sma - smab, smb - smab
        i0 = sm - sma - smb + smab
        inter = jnp.stack([i0, i1, i2, i3], axis=1)
        card = (jnp.stack([p0, p1, p2, p3], axis=1)
                + jnp.stack([t0, t1, t2, t3], axis=1))
        score_ref[...] = jnp.mean(
            2.0 * inter / jnp.maximum(card, 1.0), axis=0, keepdims=True)


def kernel(output, target):
    _, score = pl.pallas_call(
        _dice_body,
        grid=(_NSTEPS,),
        in_specs=[
            pl.BlockSpec((8, 4, _R, 512), lambda i: (0, 0, i, 0)),
            pl.BlockSpec((8, 1, _R, 512), lambda i: (0, 0, i, 0)),
        ],
        out_specs=[
            pl.BlockSpec((10, 8, 512), lambda i: (0, 0, 0)),
            pl.BlockSpec((1, 4), lambda i: (0, 0)),
        ],
        out_shape=[
            jax.ShapeDtypeStruct((10, 8, 512), jnp.int32),
            jax.ShapeDtypeStruct((1, 4), jnp.float32),
        ],
    )(output, target)
    return score[0]


# speedup vs baseline: 1.1110x; 1.1110x over previous
import jax
import jax.numpy as jnp
from jax.experimental import pallas as pl

_NPIX = float(512 * 512)
_RC = 8
_NCHUNK = 512 // _RC

_G = [0,
      (1 << 8) + (1 << 24),
      1 + (2 << 24),
      1 + (1 << 8) + (1 << 16) + (3 << 24)]
_H = [1,
      1 + (1 << 16),
      1 + (1 << 8),
      1 + (1 << 8) + (1 << 16) + (1 << 24)]


def _dice_body(o_ref, t_ref, acc_ref, score_ref):
    b = pl.program_id(0)
    nb = pl.num_programs(0)
    i32 = jnp.int32

    def chunk(r, carry):
        ag, am, ata, atb, atab = carry
        rs = pl.ds(r * _RC, _RC)
        o0 = o_ref[0, 0, rs, :]
        o1 = o_ref[0, 1, rs, :]
        o2 = o_ref[0, 2, rs, :]
        o3 = o_ref[0, 3, rs, :]
        t = t_ref[0, 0, rs, :]

        gt1 = o1 > o0
        gt3 = o3 > o2
        gtb = jnp.maximum(o2, o3) > jnp.maximum(o0, o1)
        g = jnp.where(gtb,
                      jnp.where(gt3, i32(_G[3]), i32(_G[2])),
                      jnp.where(gt1, i32(_G[1]), i32(_G[0])))
        h = jnp.where(gtb,
                      jnp.where(gt3, i32(_H[3]), i32(_H[2])),
                      jnp.where(gt1, i32(_H[1]), i32(_H[0])))
        mp = jnp.where((g >> 24) == t, h, i32(0))
        ta = t >> 1
        tb = t & 1
        return (ag + g, am + mp, ata + ta, atb + tb, atab + (ta & tb))

    zeros = jnp.zeros((_RC, 512), jnp.int32)
    ag, am, ata, atb, atab = jax.lax.fori_loop(
        0, _NCHUNK, chunk, (zeros, zeros, zeros, zeros, zeros))

    m8 = i32(0xFF)
    acc_ref[0, b] = ag & m8
    acc_ref[1, b] = (ag >> 8) & m8
    acc_ref[2, b] = (ag >> 16) & m8
    acc_ref[3, b] = ata
    acc_ref[4, b] = atb
    acc_ref[5, b] = atab
    acc_ref[6, b] = am & m8
    acc_ref[7, b] = (am >> 8) & m8
    acc_ref[8, b] = (am >> 16) & m8
    acc_ref[9, b] = (am >> 24) & m8

    @pl.when(b == nb - 1)
    def _():
        st = jnp.sum(acc_ref[...], axis=(2, 3)).astype(jnp.float32)
        sa, sb, sab = st[0], st[1], st[2]
        sta, stb, stab = st[3], st[4], st[5]
        sm, sma, smb, smab = st[6], st[7], st[8], st[9]
        p3, p2, p1 = sab, sa - sab, sb - sab
        p0 = _NPIX - sa - sb + sab
        t3, t2, t1 = stab, sta - stab, stb - stab
        t0 = _NPIX - sta - stb + stab
        i3, i2, i1 = smab, sma - smab, smb - smab
        i0 = sm - sma - smb + smab
        inter = jnp.stack([i0, i1, i2, i3], axis=1)
        card = (jnp.stack([p0, p1, p2, p3], axis=1)
                + jnp.stack([t0, t1, t2, t3], axis=1))
        score_ref[...] = jnp.mean(
            2.0 * inter / jnp.maximum(card, 1.0), axis=0, keepdims=True)


def kernel(output, target):
    _, score = pl.pallas_call(
        _dice_body,
        grid=(8,),
        in_specs=[
            pl.BlockSpec((1, 4, 512, 512), lambda i: (i, 0, 0, 0)),
            pl.BlockSpec((1, 1, 512, 512), lambda i: (i, 0, 0, 0)),
        ],
        out_specs=[
            pl.BlockSpec((10, 8, _RC, 512), lambda i: (0, 0, 0, 0)),
            pl.BlockSpec((1, 4), lambda i: (0, 0)),
        ],
        out_shape=[
            jax.ShapeDtypeStruct((10, 8, _RC, 512), jnp.int32),
            jax.ShapeDtypeStruct((1, 4), jnp.float32),
        ],
    )(output, target)
    return score[0]


# speedup vs baseline: 1.1893x; 1.0704x over previous
import jax
import jax.numpy as jnp
from jax.experimental import pallas as pl

_NPIX = float(512 * 512)
_RC = 8
_NCHUNK = 512 // _RC

_G = [0,
      (1 << 8) + (1 << 24),
      1 + (2 << 24),
      1 + (1 << 8) + (1 << 16) + (3 << 24)]
_H = [1,
      1 + (1 << 16),
      1 + (1 << 8),
      1 + (1 << 8) + (1 << 16) + (1 << 24)]


def _dice_body(o_ref, t_ref, acc_ref, score_ref):
    b = pl.program_id(0)
    nb = pl.num_programs(0)
    i32 = jnp.int32

    def half(r, u, carry):
        ag, am, ata, atb, atab = carry
        rs = pl.ds(r * (2 * _RC) + u * _RC, _RC)
        o0 = o_ref[0, 0, rs, :]
        o1 = o_ref[0, 1, rs, :]
        o2 = o_ref[0, 2, rs, :]
        o3 = o_ref[0, 3, rs, :]
        t = t_ref[0, 0, rs, :]

        gt1 = o1 > o0
        gt3 = o3 > o2
        gtb = jnp.maximum(o2, o3) > jnp.maximum(o0, o1)
        g = jnp.where(gtb,
                      jnp.where(gt3, i32(_G[3]), i32(_G[2])),
                      jnp.where(gt1, i32(_G[1]), i32(_G[0])))
        h = jnp.where(gtb,
                      jnp.where(gt3, i32(_H[3]), i32(_H[2])),
                      jnp.where(gt1, i32(_H[1]), i32(_H[0])))
        mp = jnp.where((g >> 24) == t, h, i32(0))
        ta = t >> 1
        tb = t & 1
        return (ag + g, am + mp, ata + ta, atb + tb, atab + (ta & tb))

    def chunk(r, carry):
        return half(r, 1, half(r, 0, carry))

    zeros = jnp.zeros((_RC, 512), jnp.int32)
    ag, am, ata, atb, atab = jax.lax.fori_loop(
        0, _NCHUNK // 2, chunk, (zeros, zeros, zeros, zeros, zeros))

    m8 = i32(0xFF)
    acc_ref[0, b] = ag & m8
    acc_ref[1, b] = (ag >> 8) & m8
    acc_ref[2, b] = (ag >> 16) & m8
    acc_ref[3, b] = ata
    acc_ref[4, b] = atb
    acc_ref[5, b] = atab
    acc_ref[6, b] = am & m8
    acc_ref[7, b] = (am >> 8) & m8
    acc_ref[8, b] = (am >> 16) & m8
    acc_ref[9, b] = (am >> 24) & m8

    @pl.when(b == nb - 1)
    def _():
        st = jnp.sum(acc_ref[...], axis=(2, 3)).astype(jnp.float32)
        sa, sb, sab = st[0], st[1], st[2]
        sta, stb, stab = st[3], st[4], st[5]
        sm, sma, smb, smab = st[6], st[7], st[8], st[9]
        p3, p2, p1 = sab, sa - sab, sb - sab
        p0 = _NPIX - sa - sb + sab
        t3, t2, t1 = stab, sta - stab, stb - stab
        t0 = _NPIX - sta - stb + stab
        i3, i2, i1 = smab, sma - smab, smb - smab
        i0 = sm - sma - smb + smab
        inter = jnp.stack([i0, i1, i2, i3], axis=1)
        card = (jnp.stack([p0, p1, p2, p3], axis=1)
                + jnp.stack([t0, t1, t2, t3], axis=1))
        score_ref[...] = jnp.mean(
            2.0 * inter / jnp.maximum(card, 1.0), axis=0, keepdims=True)


def kernel(output, target):
    _, score = pl.pallas_call(
        _dice_body,
        grid=(8,),
        in_specs=[
            pl.BlockSpec((1, 4, 512, 512), lambda i: (i, 0, 0, 0)),
            pl.BlockSpec((1, 1, 512, 512), lambda i: (i, 0, 0, 0)),
        ],
        out_specs=[
            pl.BlockSpec((10, 8, _RC, 512), lambda i: (0, 0, 0, 0)),
            pl.BlockSpec((1, 4), lambda i: (0, 0)),
        ],
        out_shape=[
            jax.ShapeDtypeStruct((10, 8, _RC, 512), jnp.int32),
            jax.ShapeDtypeStruct((1, 4), jnp.float32),
        ],
    )(output, target)
    return score[0]
